# baseline (speedup 1.0000x reference)
import jax
import jax.numpy as jnp
from jax import lax
from jax.experimental import pallas as pl
from jax.experimental.pallas import tpu as pltpu
from jax.experimental.pallas import tpu_sc as plsc

N_NODES = 10000
N_EDGES = 160000
D = 256
HALF = 128
EPS1 = 1.5

NUM_SC = 2
NUM_SUBCORES = 16
BATCH = 128
NB = 80
SLAB = 40
E_PER_TILE = NB * BATCH
E_PAD = NUM_SUBCORES * E_PER_TILE
ACC_ROWS = 10240
ZSTRIPE = ACC_ROWS // NUM_SUBCORES


def _sc_agg_kernel(xs_hbm, src_hbm, dst_hbm, zeros_hbm, out_hbm, acc, src_v,
                   dst_v, rows_a, rows_b, sem_ga, sem_gb, sem_aa, sem_ab):
    c = lax.axis_index("c")
    s = lax.axis_index("s")

    pltpu.sync_copy(zeros_hbm, rows_a)
    for k in range(ZSTRIPE // BATCH):
        pltpu.sync_copy(rows_a, acc.at[pl.ds(s * ZSTRIPE + k * BATCH, BATCH)])
    rem = ZSTRIPE % BATCH
    if rem:
        pltpu.sync_copy(
            rows_a.at[pl.ds(0, rem)],
            acc.at[pl.ds(s * ZSTRIPE + (ZSTRIPE // BATCH) * BATCH, rem)])

    plsc.subcore_barrier()

    pltpu.sync_copy(src_hbm.at[s], src_v)

    def gth(j, buf, sem):
        return pltpu.make_async_copy(
            xs_hbm.at[src_v.at[j], pl.ds(c * HALF, HALF)], buf, sem)

    class add:
        def __init__(self, jj, buf, sem):
            self.args = (buf, acc.at[dst_v.at[jj]], sem)

        def start(self):
            pltpu.async_copy(*self.args, add=True)

        def wait(self):
            pltpu.make_async_copy(*self.args).wait()

    for g in range(NB // SLAB):
        pltpu.sync_copy(dst_hbm.at[s, pl.ds(g * SLAB, SLAB)], dst_v)

        @pl.loop(0, SLAB, step=2)
        def _(jj):
            j = g * SLAB + jj
            gth(j, rows_a, sem_ga).start()
            gth(j + 1, rows_b, sem_gb).start()
            gth(j, rows_a, sem_ga).wait()
            gth(j + 1, rows_b, sem_gb).wait()
            pltpu.sync_copy(rows_a, acc.at[dst_v.at[jj]], add=True)
            pltpu.sync_copy(rows_b, acc.at[dst_v.at[jj + 1]], add=True)

    plsc.subcore_barrier()

    pltpu.sync_copy(acc.at[pl.ds(s * ZSTRIPE, ZSTRIPE)],
                    out_hbm.at[c, pl.ds(s * ZSTRIPE, ZSTRIPE)])


def _sc_aggregate(xs, srcs, dsts):
    mesh = plsc.VectorSubcoreMesh(core_axis_name="c", subcore_axis_name="s")
    kern = pl.kernel(
        _sc_agg_kernel,
        out_type=jax.ShapeDtypeStruct((NUM_SC, ACC_ROWS, HALF), jnp.float32),
        mesh=mesh,
        scratch_types=[
            pltpu.VMEM_SHARED((ACC_ROWS, HALF), jnp.float32),
            pltpu.VMEM((NB, BATCH), jnp.int32),
            pltpu.VMEM((SLAB, BATCH), jnp.int32),
            pltpu.VMEM((BATCH, HALF), jnp.float32),
            pltpu.VMEM((BATCH, HALF), jnp.float32),
            pltpu.SemaphoreType.DMA,
            pltpu.SemaphoreType.DMA,
            pltpu.SemaphoreType.DMA,
            pltpu.SemaphoreType.DMA,
        ],
    )
    zeros = jnp.zeros((BATCH, HALF), jnp.float32)
    return kern(xs, srcs, dsts, zeros)


def _mm_body(x_ref, lo_ref, hi_ref, w_ref, o_ref):
    agg = jnp.concatenate([lo_ref[0], hi_ref[0]], axis=-1)
    xa = EPS1 * x_ref[...] + agg
    o_ref[...] = jnp.dot(xa, w_ref[...], preferred_element_type=jnp.float32)


def _tc_linear(x, agg_pad, W):
    rows = 1000
    grid = (N_NODES // rows,)
    return pl.pallas_call(
        _mm_body,
        grid=grid,
        in_specs=[
            pl.BlockSpec((rows, D), lambda i: (i, 0)),
            pl.BlockSpec((1, rows, HALF), lambda i: (0, i, 0)),
            pl.BlockSpec((1, rows, HALF), lambda i: (1, i, 0)),
            pl.BlockSpec((D, D), lambda i: (0, 0)),
        ],
        out_specs=pl.BlockSpec((rows, D), lambda i: (i, 0)),
        out_shape=jax.ShapeDtypeStruct((N_NODES, D), jnp.float32),
    )(x, agg_pad, agg_pad, W)


def kernel(x, edge_index, W):
    src = edge_index[0].astype(jnp.int32)
    dst = edge_index[1].astype(jnp.int32)

    pad = E_PAD - N_EDGES
    src_p = jnp.concatenate([src, jnp.zeros((pad,), jnp.int32)])
    dst_p = jnp.concatenate([dst, jnp.full((pad,), N_NODES, jnp.int32)])
    srcs = src_p.reshape(NUM_SUBCORES, NB, BATCH)
    dsts = dst_p.reshape(NUM_SUBCORES, NB, BATCH)

    agg_pad = _sc_aggregate(x, srcs, dsts)
    return _tc_linear(x, agg_pad, W)


# speedup vs baseline: 1.2978x; 1.2978x over previous
import jax
import jax.numpy as jnp
from jax import lax
from jax.experimental import pallas as pl
from jax.experimental.pallas import tpu as pltpu
from jax.experimental.pallas import tpu_sc as plsc

N_NODES = 10000
N_EDGES = 160000
D = 256
HALF = 128
EPS1 = 1.5

NUM_SC = 2
NUM_SUBCORES = 16
BATCH = 128
NB = 79
SLAB = 40
E_PER_TILE = NB * BATCH
E_PAD = NUM_SUBCORES * E_PER_TILE
ACC_ROWS = 10240
ZSTRIPE = ACC_ROWS // NUM_SUBCORES


def _sc_agg_kernel(xs_hbm, src_hbm, dst_hbm, zeros_hbm, out_hbm, acc, src_v,
                   dst_v, rows_a, sem_ga):
    c = lax.axis_index("c")
    s = lax.axis_index("s")

    pltpu.sync_copy(zeros_hbm, rows_a)
    for k in range(ZSTRIPE // BATCH):
        pltpu.sync_copy(rows_a, acc.at[pl.ds(s * ZSTRIPE + k * BATCH, BATCH)])
    rem = ZSTRIPE % BATCH
    if rem:
        pltpu.sync_copy(
            rows_a.at[pl.ds(0, rem)],
            acc.at[pl.ds(s * ZSTRIPE + (ZSTRIPE // BATCH) * BATCH, rem)])

    plsc.subcore_barrier()

    pltpu.sync_copy(src_hbm.at[s], src_v)
    pltpu.sync_copy(dst_hbm.at[s], dst_v)

    @pl.loop(0, NB)
    def _(j):
        pltpu.async_copy(xs_hbm.at[src_v.at[j], pl.ds(c * HALF, HALF)],
                         rows_a, sem_ga).wait()
        pltpu.sync_copy(rows_a, acc.at[dst_v.at[j]], add=True)

    plsc.subcore_barrier()

    pltpu.sync_copy(acc.at[pl.ds(s * ZSTRIPE, ZSTRIPE)],
                    out_hbm.at[c, pl.ds(s * ZSTRIPE, ZSTRIPE)])


def _sc_aggregate(xs, srcs, dsts):
    mesh = plsc.VectorSubcoreMesh(core_axis_name="c", subcore_axis_name="s")
    kern = pl.kernel(
        _sc_agg_kernel,
        out_type=jax.ShapeDtypeStruct((NUM_SC, ACC_ROWS, HALF), jnp.float32),
        mesh=mesh,
        scratch_types=[
            pltpu.VMEM_SHARED((ACC_ROWS, HALF), jnp.float32),
            pltpu.VMEM((NB, BATCH), jnp.int32),
            pltpu.VMEM((NB, BATCH), jnp.int32),
            pltpu.VMEM((BATCH, HALF), jnp.float32),
            pltpu.SemaphoreType.DMA,
        ],
    )
    zeros = jnp.zeros((BATCH, HALF), jnp.float32)
    return kern(xs, srcs, dsts, zeros)


def _mm_body(x_ref, lo_ref, hi_ref, w_ref, o_ref):
    agg = jnp.concatenate([lo_ref[0], hi_ref[0]], axis=-1)
    xa = EPS1 * x_ref[...] + agg
    o_ref[...] = jnp.dot(xa, w_ref[...], preferred_element_type=jnp.float32)


def _tc_linear(x, agg_pad, W):
    rows = 1000
    grid = (N_NODES // rows,)
    return pl.pallas_call(
        _mm_body,
        grid=grid,
        in_specs=[
            pl.BlockSpec((rows, D), lambda i: (i, 0)),
            pl.BlockSpec((1, rows, HALF), lambda i: (0, i, 0)),
            pl.BlockSpec((1, rows, HALF), lambda i: (1, i, 0)),
            pl.BlockSpec((D, D), lambda i: (0, 0)),
        ],
        out_specs=pl.BlockSpec((rows, D), lambda i: (i, 0)),
        out_shape=jax.ShapeDtypeStruct((N_NODES, D), jnp.float32),
    )(x, agg_pad, agg_pad, W)


def kernel(x, edge_index, W):
    src = edge_index[0].astype(jnp.int32)
    dst = edge_index[1].astype(jnp.int32)

    pad = E_PAD - N_EDGES
    src_p = jnp.concatenate([src, jnp.zeros((pad,), jnp.int32)])
    dst_p = jnp.concatenate([dst, jnp.full((pad,), N_NODES, jnp.int32)])
    srcs = src_p.reshape(NUM_SUBCORES, NB, BATCH)
    dsts = dst_p.reshape(NUM_SUBCORES, NB, BATCH)

    agg_pad = _sc_aggregate(x, srcs, dsts)
    return _tc_linear(x, agg_pad, W)
